# baseline (speedup 1.0000x reference)
import functools

import jax
import jax.numpy as jnp
from jax import lax
from jax.experimental import pallas as pl
from jax.experimental.pallas import tpu as pltpu
from jax.experimental.pallas import tpu_sc as plsc

_N, _D, _E = 8192, 1024, 4
_NC, _NS, _L = 2, 16, 16
_NT = _NC * _NS
_RANGE = 1024
_NR = _N // _RANGE
_NSEG = _E * _NR
_CAP = _RANGE
_ZROW = _NSEG * _CAP
_G = 32
_BLKF = 256
_CTOK = _N // _NT


def _gate_body(x_ref, wgn_ref, thr_ref, mask_ref, noise_ref,
               gt_ref, pt_ref, cnt_ref):
    i = pl.program_id(0)
    x = x_ref[...]
    logits = jnp.dot(x, wgn_ref[...], preferred_element_type=jnp.float32)
    clean = logits[:, :_E]
    raw_noise = logits[:, _E:]
    noise_std = jax.nn.softplus(raw_noise) + 1e-2
    noisy = clean + noise_ref[...] * noise_std
    scores = noisy - thr_ref[...]
    signed = jnp.sign(scores)
    sel = 0.5 * (signed + 1.0) * mask_ref[...]
    masked = jnp.where(sel > 0.0, clean, jnp.full_like(clean, -1e9))
    m = jnp.max(masked, axis=-1, keepdims=True)
    ex = jnp.exp(masked - m)
    gates = (ex / jnp.sum(ex, axis=-1, keepdims=True)) * sel
    denom = jnp.clip(jnp.sum(gates, axis=-1, keepdims=True), 1e-9, None)
    gates = gates / denom
    gt_ref[...] = jnp.transpose(gates)

    seli = jnp.where(sel > 0.0, 1.0, 0.0)
    ri = lax.broadcasted_iota(jnp.int32, (_RANGE, _RANGE), 0)
    ci = lax.broadcasted_iota(jnp.int32, (_RANGE, _RANGE), 1)
    tri = jnp.where(ri >= ci, 1.0, 0.0)
    csum = jnp.dot(tri, seli, precision=lax.Precision.HIGHEST,
                   preferred_element_type=jnp.float32)
    rank = (csum - seli).astype(jnp.int32)
    col = lax.broadcasted_iota(jnp.int32, (_RANGE, _E), 1)
    seg_base = (col * _NR + i) * _CAP
    pos = jnp.where(sel > 0.0, seg_base + rank, _ZROW)
    pt_ref[...] = jnp.transpose(pos)
    cnt_ref[...] = csum[_RANGE - 1:_RANGE, :].astype(jnp.int32).reshape(1, 1, _E)


def _gating(x, wgn, thr, mask, noise):
    return pl.pallas_call(
        _gate_body,
        grid=(_NR,),
        in_specs=[
            pl.BlockSpec((_RANGE, _D), lambda i: (i, 0)),
            pl.BlockSpec((_D, 2 * _E), lambda i: (0, 0)),
            pl.BlockSpec((1, _E), lambda i: (0, 0)),
            pl.BlockSpec((1, _E), lambda i: (0, 0)),
            pl.BlockSpec((_RANGE, _E), lambda i: (i, 0)),
        ],
        out_specs=[
            pl.BlockSpec((_E, _RANGE), lambda i: (0, i)),
            pl.BlockSpec((_E, _RANGE), lambda i: (0, i)),
            pl.BlockSpec((1, 1, _E), lambda i: (i, 0, 0)),
        ],
        out_shape=[
            jax.ShapeDtypeStruct((_E, _N), jnp.float32),
            jax.ShapeDtypeStruct((_E, _N), jnp.int32),
            jax.ShapeDtypeStruct((_NR, 1, _E), jnp.int32),
        ],
    )(x, wgn, thr, mask, noise)


_MESH = plsc.VectorSubcoreMesh(core_axis_name="c", subcore_axis_name="s")


@functools.partial(
    pl.kernel,
    out_type=(
        jax.ShapeDtypeStruct((_NSEG * _CAP, _D), jnp.float32),
        jax.ShapeDtypeStruct((_NSEG * _CAP,), jnp.float32),
    ),
    mesh=_MESH,
    scratch_types=[
        pltpu.VMEM((_RANGE,), jnp.int32),
        pltpu.VMEM((_RANGE,), jnp.float32),
        pltpu.VMEM((_RANGE + 2 * _L,), jnp.int32),
        pltpu.VMEM((_RANGE + 2 * _L,), jnp.float32),
        pltpu.VMEM((_G, _D), jnp.float32),
        pltpu.SemaphoreType.DMA,
    ],
    compiler_params=pltpu.CompilerParams(needs_layout_passes=False),
)
def _sc_compact_gather(pos_hbm, gate_hbm, x_hbm, xg, gc,
                       pos_v, gate_v, idx_v, gc_v, rows_v, sem):
    c = lax.axis_index("c")
    s = lax.axis_index("s")
    e = c * (_E // _NC) + s // _NR
    r = s % _NR
    tok0 = r * _RANGE
    seg = e * _NR + r
    seg0 = seg * _CAP

    pltpu.sync_copy(pos_hbm.at[pl.ds(e * _N + tok0, _RANGE)], pos_v)
    pltpu.sync_copy(gate_hbm.at[pl.ds(e * _N + tok0, _RANGE)], gate_v)

    iota = lax.iota(jnp.int32, _L)
    zrow_v = jnp.full((_L,), _ZROW, jnp.int32)
    seg0_v = jnp.full((_L,), seg0, jnp.int32)
    dump_v = jnp.full((_L,), _RANGE, jnp.int32)

    def cbody(j, cnt_s):
        p = pos_v[pl.ds(j * _L, _L)]
        g = gate_v[pl.ds(j * _L, _L)]
        msk = p < zrow_v
        ids = jnp.full((_L,), tok0 + j * _L, jnp.int32) + iota
        dst = jnp.where(msk, p - seg0_v, dump_v)
        plsc.store_scatter(idx_v, [dst], ids)
        plsc.store_scatter(gc_v, [dst], g)
        pc = plsc.all_reduce_population_count(msk)
        return cnt_s + pc[0]

    cnt_s = lax.fori_loop(0, _RANGE // _L, cbody, 0)

    idx_v[pl.ds(cnt_s, _L)] = jnp.zeros((_L,), jnp.int32)
    idx_v[pl.ds(cnt_s + _L, _L)] = jnp.zeros((_L,), jnp.int32)

    pltpu.sync_copy(gc_v.at[pl.ds(0, _RANGE)], gc.at[pl.ds(seg0, _RANGE)])

    nch = (cnt_s + _G - 1) // _G

    def gbody(i, carry):
        pltpu.async_copy(
            x_hbm.at[idx_v.at[pl.ds(i * _G, _G)]], rows_v, sem).wait()
        pltpu.sync_copy(rows_v, xg.at[pl.ds(seg0 + i * _G, _G)])
        return carry

    lax.fori_loop(0, nch, gbody, 0)


def _ffn_body(cnt_ref, xg_ref, gc_ref, W1_ref, b1_ref, W2_ref, b2_ref, yg_ref):
    t = pl.program_id(0)
    nblk = _CAP // _BLKF

    @pl.when(t == _NSEG * nblk)
    def _zero():
        yg_ref[...] = jnp.zeros_like(yg_ref)

    @pl.when(t < _NSEG * nblk)
    def _compute():
        seg = t // nblk
        j = t % nblk

        @pl.when(j * _BLKF < cnt_ref[seg])
        def _():
            xb = xg_ref[...]
            gcb = gc_ref[...]
            h = jnp.dot(xb, W1_ref[0], preferred_element_type=jnp.float32)
            h = jnp.maximum(h + b1_ref[0], 0.0) * gcb
            y = jnp.dot(h, W2_ref[0], preferred_element_type=jnp.float32)
            yg_ref[...] = y + gcb * b2_ref[0]


def _ffn(cnt, xg, gc2d, W1, b1, W2, b2):
    nblk = _CAP // _BLKF
    return pl.pallas_call(
        _ffn_body,
        grid_spec=pltpu.PrefetchScalarGridSpec(
            num_scalar_prefetch=1,
            grid=(_NSEG * nblk + 1,),
            in_specs=[
                pl.BlockSpec((_BLKF, _D),
                             lambda t, cnt: (jnp.minimum(t, _NSEG * nblk - 1), 0)),
                pl.BlockSpec((_BLKF, 1),
                             lambda t, cnt: (jnp.minimum(t, _NSEG * nblk - 1), 0)),
                pl.BlockSpec((1, _D, _D),
                             lambda t, cnt: (jnp.minimum(t // (_NR * nblk), _E - 1), 0, 0)),
                pl.BlockSpec((1, 1, _D),
                             lambda t, cnt: (jnp.minimum(t // (_NR * nblk), _E - 1), 0, 0)),
                pl.BlockSpec((1, _D, _D),
                             lambda t, cnt: (jnp.minimum(t // (_NR * nblk), _E - 1), 0, 0)),
                pl.BlockSpec((1, 1, _D),
                             lambda t, cnt: (jnp.minimum(t // (_NR * nblk), _E - 1), 0, 0)),
            ],
            out_specs=pl.BlockSpec((_BLKF, _D), lambda t, cnt: (t, 0)),
        ),
        out_shape=jax.ShapeDtypeStruct(((_NSEG * nblk + 1) * _BLKF, _D),
                                       jnp.float32),
        compiler_params=pltpu.CompilerParams(
            dimension_semantics=("arbitrary",),
        ),
    )(cnt, xg, gc2d, W1, b1.reshape(_E, 1, _D), W2, b2.reshape(_E, 1, _D))


@functools.partial(
    pl.kernel,
    out_type=jax.ShapeDtypeStruct((_N, _D), jnp.float32),
    mesh=_MESH,
    scratch_types=[
        pltpu.VMEM((_E * _CTOK,), jnp.int32),
        pltpu.VMEM((_L, _D), jnp.float32),
        pltpu.VMEM((_L, _D), jnp.float32),
        pltpu.VMEM((_L, _D), jnp.float32),
        pltpu.VMEM((_L, _D), jnp.float32),
        pltpu.VMEM((_L, _D), jnp.float32),
        pltpu.SemaphoreType.DMA,
        pltpu.SemaphoreType.DMA,
        pltpu.SemaphoreType.DMA,
        pltpu.SemaphoreType.DMA,
    ],
)
def _sc_combine(pos_hbm, yg_hbm, out_hbm,
                pos_v, buf0, buf1, buf2, buf3, acc, s0, s1, s2, s3):
    c = lax.axis_index("c")
    s = lax.axis_index("s")
    wid = c * _NS + s
    tok0 = wid * _CTOK

    for e in range(_E):
        pltpu.sync_copy(pos_hbm.at[pl.ds(e * _N + tok0, _CTOK)],
                        pos_v.at[pl.ds(e * _CTOK, _CTOK)])

    bufs = (buf0, buf1, buf2, buf3)
    sems = (s0, s1, s2, s3)

    def tbody(t, carry):
        waits = []
        for e in range(_E):
            waits.append(pltpu.async_copy(
                yg_hbm.at[pos_v.at[pl.ds(e * _CTOK + t * _L, _L)]],
                bufs[e], sems[e]))
        for w in waits:
            w.wait()

        def rbody(q):
            row = q // (_D // _L)
            col = (q % (_D // _L)) * _L
            acc[row, pl.ds(col, _L)] = (
                buf0[row, pl.ds(col, _L)] + buf1[row, pl.ds(col, _L)]
                + buf2[row, pl.ds(col, _L)] + buf3[row, pl.ds(col, _L)])

        plsc.parallel_loop(0, _L * (_D // _L), 1, unroll=8)(rbody)
        pltpu.sync_copy(acc, out_hbm.at[pl.ds(tok0 + t * _L, _L)])
        return carry

    lax.fori_loop(0, _CTOK // _L, tbody, 0)


def kernel(x, w_gate, w_noise, gate_threshold, experts_mask, noise, W1, b1, W2, b2):
    wgn = jnp.concatenate([w_gate, w_noise], axis=1)
    thr = gate_threshold.reshape(1, _E)
    mask = experts_mask.reshape(1, _E)

    gates_t, pos_t, cnt_nr = _gating(x, wgn, thr, mask, noise)
    pos = pos_t.reshape(-1)
    cnt = jnp.transpose(cnt_nr.reshape(_NR, _E)).reshape(_NSEG)
    xg, gc = _sc_compact_gather(pos, gates_t.reshape(-1), x)
    yg = _ffn(cnt, xg, gc.reshape(-1, 1), W1, b1, W2, b2)
    out = _sc_combine(pos, yg)
    return out


# speedup vs baseline: 6.6694x; 6.6694x over previous
import jax
import jax.numpy as jnp
from jax.experimental import pallas as pl
from jax.experimental.pallas import tpu as pltpu

_N, _D, _E = 8192, 1024, 4
_BLK = 512


def _fused_body(x_ref, wgn_ref, thr_ref, mask_ref, noise_ref,
                W1_ref, b1_ref, W2_ref, b2_ref, out_ref):
    x = x_ref[...]
    logits = jnp.dot(x, wgn_ref[...], preferred_element_type=jnp.float32)
    clean = logits[:, :_E]
    raw_noise = logits[:, _E:]
    noise_std = jax.nn.softplus(raw_noise) + 1e-2
    noisy = clean + noise_ref[...] * noise_std
    scores = noisy - thr_ref[...]
    signed = jnp.sign(scores)
    sel = 0.5 * (signed + 1.0) * mask_ref[...]
    masked = jnp.where(sel > 0.0, clean, jnp.full_like(clean, -1e9))
    m = jnp.max(masked, axis=-1, keepdims=True)
    ex = jnp.exp(masked - m)
    gates = (ex / jnp.sum(ex, axis=-1, keepdims=True)) * sel
    denom = jnp.clip(jnp.sum(gates, axis=-1, keepdims=True), 1e-9, None)
    gates = gates / denom

    acc = jnp.zeros_like(x)
    for e in range(_E):
        h = jnp.dot(x, W1_ref[e], preferred_element_type=jnp.float32)
        h = jnp.maximum(h + b1_ref[e:e + 1, :], 0.0)
        y = jnp.dot(h, W2_ref[e], preferred_element_type=jnp.float32)
        y = y + b2_ref[e:e + 1, :]
        acc = acc + gates[:, e:e + 1] * y
    out_ref[...] = acc


def kernel(x, w_gate, w_noise, gate_threshold, experts_mask, noise, W1, b1, W2, b2):
    wgn = jnp.concatenate([w_gate, w_noise], axis=1)
    thr = gate_threshold.reshape(1, _E)
    mask = experts_mask.reshape(1, _E)

    grid = (_N // _BLK,)
    out = pl.pallas_call(
        _fused_body,
        grid=grid,
        in_specs=[
            pl.BlockSpec((_BLK, _D), lambda i: (i, 0)),
            pl.BlockSpec((_D, 2 * _E), lambda i: (0, 0)),
            pl.BlockSpec((1, _E), lambda i: (0, 0)),
            pl.BlockSpec((1, _E), lambda i: (0, 0)),
            pl.BlockSpec((_BLK, _E), lambda i: (i, 0)),
            pl.BlockSpec((_E, _D, _D), lambda i: (0, 0, 0)),
            pl.BlockSpec((_E, _D), lambda i: (0, 0)),
            pl.BlockSpec((_E, _D, _D), lambda i: (0, 0, 0)),
            pl.BlockSpec((_E, _D), lambda i: (0, 0)),
        ],
        out_specs=pl.BlockSpec((_BLK, _D), lambda i: (i, 0)),
        out_shape=jax.ShapeDtypeStruct((_N, _D), jnp.float32),
        compiler_params=pltpu.CompilerParams(
            dimension_semantics=("parallel",),
        ),
    )(x, wgn, thr, mask, noise, W1, b1, W2, b2)
    return out
